# Initial kernel scaffold; baseline (speedup 1.0000x reference)
import jax
import jax.numpy as jnp
from jax.experimental import pallas as pl


def kernel(x, edge_index, W1_l, b1_l, W1_r, Wmu_l, bmu_l, Wmu_r, Wls_l, bls_l, Wls_r):
    raise NotImplementedError("write your pallas kernel here")


# speedup vs baseline: 3.4034x; 3.4034x over previous
import functools

import jax
import jax.numpy as jnp
from jax import lax
from jax.experimental import pallas as pl
from jax.experimental.pallas import tpu as pltpu
from jax.experimental.pallas import tpu_sc as plsc

_N = 10000
_NP = 10240
_E = 160000
_F = 256
_HF = 128
_NC = 2
_NS = 16
_CH = 80
_EPS = _E // _NS
_NCHUNK = _EPS // _CH
_CHC = 40
_EPW = _E // (_NC * _NS)
_NCHUNKC = _EPW // _CHC
_RPS = _NP // _NS
_ZR = 128
_RB = 1000


def _mesh():
    return plsc.VectorSubcoreMesh(
        core_axis_name="c", subcore_axis_name="s",
        num_cores=_NC, num_subcores=_NS)


def _make_seg_sum():
    out_type = (
        jax.ShapeDtypeStruct((_NP, _HF), jnp.float32),
        jax.ShapeDtypeStruct((_NP, _HF), jnp.float32),
    )
    scratch = (
        pltpu.VMEM_SHARED((_NP, _HF), jnp.float32),
        pltpu.VMEM((_ZR, _HF), jnp.float32),
        pltpu.VMEM((_CH,), jnp.int32),
        pltpu.VMEM((_CH,), jnp.int32),
        pltpu.VMEM((_CH, _HF), jnp.float32),
        pltpu.SemaphoreType.DMA,
    )

    def body(src_h, dst_h, tlo_h, thi_h, z_h, slo_h, shi_h,
             acc, zbuf, src_v, dst_v, rows_v, sem):
        cid = lax.axis_index("c")
        sid = lax.axis_index("s")
        r0 = sid * _RPS

        pltpu.sync_copy(z_h, zbuf)
        for j in range(_RPS // _ZR):
            pltpu.sync_copy(zbuf, acc.at[pl.ds(r0 + j * _ZR, _ZR)])
        plsc.subcore_barrier()

        e0 = sid * _EPS

        def run(tab_h):
            def chunk(c, carry):
                off = e0 + c * _CH
                pltpu.sync_copy(src_h.at[pl.ds(off, _CH)], src_v)
                pltpu.sync_copy(dst_h.at[pl.ds(off, _CH)], dst_v)
                pltpu.async_copy(tab_h.at[src_v], rows_v, sem).wait()
                pltpu.sync_copy(rows_v, acc.at[dst_v], add=True)
                return carry
            lax.fori_loop(0, _NCHUNK, chunk, 0)

        @pl.when(cid == 0)
        def _():
            run(tlo_h)

        @pl.when(cid == 1)
        def _():
            run(thi_h)

        plsc.subcore_barrier()

        @pl.when(cid == 0)
        def _():
            pltpu.sync_copy(acc.at[pl.ds(r0, _RPS)], slo_h.at[pl.ds(r0, _RPS)])

        @pl.when(cid == 1)
        def _():
            pltpu.sync_copy(acc.at[pl.ds(r0, _RPS)], shi_h.at[pl.ds(r0, _RPS)])

    return pl.kernel(body, out_type=out_type, mesh=_mesh(),
                     scratch_types=scratch)


def _make_count():
    out_type = (
        jax.ShapeDtypeStruct((_NP, _HF), jnp.float32),
        jax.ShapeDtypeStruct((_NP, _HF), jnp.float32),
    )
    scratch = (
        pltpu.VMEM_SHARED((_NP, _HF), jnp.float32),
        pltpu.VMEM((_ZR, _HF), jnp.float32),
        pltpu.VMEM((_CHC,), jnp.int32),
        pltpu.VMEM((_CHC, _HF), jnp.float32),
    )

    def body(dst_h, z_h, o_h, p0_h, p1_h, acc, zbuf, dst_v, ones_v):
        cid = lax.axis_index("c")
        sid = lax.axis_index("s")
        r0 = sid * _RPS

        pltpu.sync_copy(z_h, zbuf)
        for j in range(_RPS // _ZR):
            pltpu.sync_copy(zbuf, acc.at[pl.ds(r0 + j * _ZR, _ZR)])
        pltpu.sync_copy(o_h, ones_v)
        plsc.subcore_barrier()

        e0 = (cid * _NS + sid) * _EPW

        def chunk(c, carry):
            off = e0 + c * _CHC
            pltpu.sync_copy(dst_h.at[pl.ds(off, _CHC)], dst_v)
            pltpu.sync_copy(ones_v, acc.at[dst_v], add=True)
            return carry
        lax.fori_loop(0, _NCHUNKC, chunk, 0)

        plsc.subcore_barrier()

        @pl.when(cid == 0)
        def _():
            pltpu.sync_copy(acc.at[pl.ds(r0, _RPS)], p0_h.at[pl.ds(r0, _RPS)])

        @pl.when(cid == 1)
        def _():
            pltpu.sync_copy(acc.at[pl.ds(r0, _RPS)], p1_h.at[pl.ds(r0, _RPS)])

    return pl.kernel(body, out_type=out_type, mesh=_mesh(),
                     scratch_types=scratch)


def _dense_body(relu, slo, shi, p0, p1, xl, xh, wl, wr, b, olo, ohi):
    c = jnp.maximum(p0[:, 0:1] + p1[:, 0:1], 1.0)
    s = jnp.concatenate([slo[...], shi[...]], axis=1)
    xx = jnp.concatenate([xl[...], xh[...]], axis=1)
    y = jnp.dot(s / c, wl[...], preferred_element_type=jnp.float32)
    y = y + jnp.dot(xx, wr[...], preferred_element_type=jnp.float32)
    y = y + b[0:1, :]
    if relu:
        y = jnp.maximum(y, 0.0)
    olo[...] = y[:, :_HF]
    ohi[...] = y[:, _HF:]


def _dense(relu, slo, shi, p0, p1, xl, xh, wl, wr, b):
    rowspec = pl.BlockSpec((_RB, _HF), lambda i: (i, 0))
    return pl.pallas_call(
        functools.partial(_dense_body, relu),
        grid=(_N // _RB,),
        in_specs=[rowspec, rowspec, rowspec, rowspec, rowspec, rowspec,
                  pl.BlockSpec((_F, _F), lambda i: (0, 0)),
                  pl.BlockSpec((_F, _F), lambda i: (0, 0)),
                  pl.BlockSpec((8, _F), lambda i: (0, 0))],
        out_specs=[rowspec, rowspec],
        out_shape=[jax.ShapeDtypeStruct((_N, _HF), jnp.float32)] * 2,
    )(slo, shi, p0, p1, xl, xh, wl, wr, b)


def kernel(x, edge_index, W1_l, b1_l, W1_r,
           Wmu_l, bmu_l, Wmu_r, Wls_l, bls_l, Wls_r):
    src = edge_index[0].astype(jnp.int32)
    dst = edge_index[1].astype(jnp.int32)
    xlo = x[:, :_HF]
    xhi = x[:, _HF:]
    z = jnp.zeros((_ZR, _HF), jnp.float32)
    ones = jnp.ones((_CHC, _HF), jnp.float32)

    seg = _make_seg_sum()
    count = _make_count()

    p0, p1 = count(dst, z, ones)
    slo, shi = seg(src, dst, xlo, xhi, z)
    b1 = jnp.tile(b1_l[None, :], (8, 1))
    hlo, hhi = _dense(True, slo, shi, p0, p1, xlo, xhi, W1_l.T, W1_r.T, b1)

    slo2, shi2 = seg(src, dst, hlo, hhi, z)
    wl2 = jnp.concatenate([Wmu_l.T, Wls_l.T], axis=1)
    wr2 = jnp.concatenate([Wmu_r.T, Wls_r.T], axis=1)
    b2 = jnp.tile(jnp.concatenate([bmu_l, bls_l])[None, :], (8, 1))
    mu, ls = _dense(False, slo2, shi2, p0, p1, hlo, hhi, wl2, wr2, b2)
    return (mu, ls)


# speedup vs baseline: 5.0075x; 1.4713x over previous
import functools

import jax
import jax.numpy as jnp
from jax import lax
from jax.experimental import pallas as pl
from jax.experimental.pallas import tpu as pltpu
from jax.experimental.pallas import tpu_sc as plsc

_N = 10000
_NP = 10240
_E = 160000
_F = 256
_HF = 128
_NC = 2
_NS = 16
_CH = 80
_EPS = _E // _NS
_NCHUNK = _EPS // _CH
_CHC = 40
_EPW = _E // (_NC * _NS)
_NCHUNKC = _EPW // _CHC
_RPS = _NP // _NS
_ZR = 128
_RB = 1000


def _mesh():
    return plsc.VectorSubcoreMesh(
        core_axis_name="c", subcore_axis_name="s",
        num_cores=_NC, num_subcores=_NS)


def _make_seg_sum():
    out_type = (
        jax.ShapeDtypeStruct((_NP, _HF), jnp.float32),
        jax.ShapeDtypeStruct((_NP, _HF), jnp.float32),
    )
    scratch = (
        pltpu.VMEM_SHARED((_NP, _HF), jnp.float32),
        pltpu.VMEM((_ZR, _HF), jnp.float32),
        pltpu.VMEM((_CH,), jnp.int32),
        pltpu.VMEM((_CH,), jnp.int32),
        pltpu.VMEM((_CH, _HF), jnp.float32),
        pltpu.SemaphoreType.DMA,
        pltpu.VMEM((_CH,), jnp.int32),
        pltpu.VMEM((_CH,), jnp.int32),
        pltpu.VMEM((_CH, _HF), jnp.float32),
        pltpu.SemaphoreType.DMA,
    )

    def body(src_h, dst_h, tlo_h, thi_h, z_h, slo_h, shi_h,
             acc, zbuf, src_v0, dst_v0, rows_v0, sem0,
             src_v1, dst_v1, rows_v1, sem1):
        cid = lax.axis_index("c")
        sid = lax.axis_index("s")
        r0 = sid * _RPS

        pltpu.sync_copy(z_h, zbuf)
        for j in range(_RPS // _ZR):
            pltpu.sync_copy(zbuf, acc.at[pl.ds(r0 + j * _ZR, _ZR)])
        plsc.subcore_barrier()

        e0 = sid * _EPS
        bufs = ((src_v0, dst_v0, rows_v0, sem0),
                (src_v1, dst_v1, rows_v1, sem1))

        def run(tab_h):
            def start(b, c):
                sv, dv, rv, sm = b
                off = e0 + c * _CH
                pltpu.sync_copy(src_h.at[pl.ds(off, _CH)], sv)
                pltpu.sync_copy(dst_h.at[pl.ds(off, _CH)], dv)
                pltpu.async_copy(tab_h.at[sv], rv, sm)

            def finish(b):
                sv, dv, rv, sm = b
                pltpu.make_async_copy(tab_h.at[sv], rv, sm).wait()
                pltpu.sync_copy(rv, acc.at[dv], add=True)

            start(bufs[0], 0)
            start(bufs[1], 1)

            def body2(c2, carry):
                for i in range(2):
                    b = bufs[i]
                    c = c2 * 2 + i
                    finish(b)
                    nxt = c + 2

                    @pl.when(nxt < _NCHUNK)
                    def _():
                        start(b, nxt)
                return carry
            lax.fori_loop(0, _NCHUNK // 2, body2, 0)
            if _NCHUNK % 2:
                finish(bufs[0])

        @pl.when(cid == 0)
        def _():
            run(tlo_h)

        @pl.when(cid == 1)
        def _():
            run(thi_h)

        plsc.subcore_barrier()

        @pl.when(cid == 0)
        def _():
            pltpu.sync_copy(acc.at[pl.ds(r0, _RPS)], slo_h.at[pl.ds(r0, _RPS)])

        @pl.when(cid == 1)
        def _():
            pltpu.sync_copy(acc.at[pl.ds(r0, _RPS)], shi_h.at[pl.ds(r0, _RPS)])

    return pl.kernel(body, out_type=out_type, mesh=_mesh(),
                     scratch_types=scratch)


def _make_count():
    out_type = (
        jax.ShapeDtypeStruct((_NP, _HF), jnp.float32),
        jax.ShapeDtypeStruct((_NP, _HF), jnp.float32),
    )
    scratch = (
        pltpu.VMEM_SHARED((_NP, _HF), jnp.float32),
        pltpu.VMEM((_ZR, _HF), jnp.float32),
        pltpu.VMEM((_CHC,), jnp.int32),
        pltpu.VMEM((_CHC, _HF), jnp.float32),
    )

    def body(dst_h, z_h, o_h, p0_h, p1_h, acc, zbuf, dst_v, ones_v):
        cid = lax.axis_index("c")
        sid = lax.axis_index("s")
        r0 = sid * _RPS

        pltpu.sync_copy(z_h, zbuf)
        for j in range(_RPS // _ZR):
            pltpu.sync_copy(zbuf, acc.at[pl.ds(r0 + j * _ZR, _ZR)])
        pltpu.sync_copy(o_h, ones_v)
        plsc.subcore_barrier()

        e0 = (cid * _NS + sid) * _EPW

        def chunk(c, carry):
            off = e0 + c * _CHC
            pltpu.sync_copy(dst_h.at[pl.ds(off, _CHC)], dst_v)
            pltpu.sync_copy(ones_v, acc.at[dst_v], add=True)
            return carry
        lax.fori_loop(0, _NCHUNKC, chunk, 0)

        plsc.subcore_barrier()

        @pl.when(cid == 0)
        def _():
            pltpu.sync_copy(acc.at[pl.ds(r0, _RPS)], p0_h.at[pl.ds(r0, _RPS)])

        @pl.when(cid == 1)
        def _():
            pltpu.sync_copy(acc.at[pl.ds(r0, _RPS)], p1_h.at[pl.ds(r0, _RPS)])

    return pl.kernel(body, out_type=out_type, mesh=_mesh(),
                     scratch_types=scratch)


def _dense_body(relu, slo, shi, p0, p1, xl, xh, wl, wr, b, olo, ohi):
    c = jnp.maximum(p0[:, 0:1] + p1[:, 0:1], 1.0)
    s = jnp.concatenate([slo[...], shi[...]], axis=1)
    xx = jnp.concatenate([xl[...], xh[...]], axis=1)
    y = jnp.dot(s / c, wl[...], preferred_element_type=jnp.float32)
    y = y + jnp.dot(xx, wr[...], preferred_element_type=jnp.float32)
    y = y + b[0:1, :]
    if relu:
        y = jnp.maximum(y, 0.0)
    olo[...] = y[:, :_HF]
    ohi[...] = y[:, _HF:]


def _dense(relu, slo, shi, p0, p1, xl, xh, wl, wr, b):
    rowspec = pl.BlockSpec((_RB, _HF), lambda i: (i, 0))
    return pl.pallas_call(
        functools.partial(_dense_body, relu),
        grid=(_N // _RB,),
        in_specs=[rowspec, rowspec, rowspec, rowspec, rowspec, rowspec,
                  pl.BlockSpec((_F, _F), lambda i: (0, 0)),
                  pl.BlockSpec((_F, _F), lambda i: (0, 0)),
                  pl.BlockSpec((8, _F), lambda i: (0, 0))],
        out_specs=[rowspec, rowspec],
        out_shape=[jax.ShapeDtypeStruct((_N, _HF), jnp.float32)] * 2,
    )(slo, shi, p0, p1, xl, xh, wl, wr, b)


def kernel(x, edge_index, W1_l, b1_l, W1_r,
           Wmu_l, bmu_l, Wmu_r, Wls_l, bls_l, Wls_r):
    src = edge_index[0].astype(jnp.int32)
    dst = edge_index[1].astype(jnp.int32)
    xlo = x[:, :_HF]
    xhi = x[:, _HF:]
    z = jnp.zeros((_ZR, _HF), jnp.float32)
    ones = jnp.ones((_CHC, _HF), jnp.float32)

    seg = _make_seg_sum()
    count = _make_count()

    p0, p1 = count(dst, z, ones)
    slo, shi = seg(src, dst, xlo, xhi, z)
    b1 = jnp.tile(b1_l[None, :], (8, 1))
    hlo, hhi = _dense(True, slo, shi, p0, p1, xlo, xhi, W1_l.T, W1_r.T, b1)

    slo2, shi2 = seg(src, dst, hlo, hhi, z)
    wl2 = jnp.concatenate([Wmu_l.T, Wls_l.T], axis=1)
    wr2 = jnp.concatenate([Wmu_r.T, Wls_r.T], axis=1)
    b2 = jnp.tile(jnp.concatenate([bmu_l, bls_l])[None, :], (8, 1))
    mu, ls = _dense(False, slo2, shi2, p0, p1, hlo, hhi, wl2, wr2, b2)
    return (mu, ls)


# speedup vs baseline: 5.3131x; 1.0610x over previous
import functools

import jax
import jax.numpy as jnp
from jax import lax
from jax.experimental import pallas as pl
from jax.experimental.pallas import tpu as pltpu
from jax.experimental.pallas import tpu_sc as plsc

_N = 10000
_NP = 10240
_E = 160000
_F = 256
_HF = 128
_NC = 2
_NS = 16
_CH = 80
_EPS = _E // _NS
_NCHUNK = _EPS // _CH
_CHC = 40
_EPW = _E // (_NC * _NS)
_NCHUNKC = _EPW // _CHC
_RPS = _NP // _NS
_ZR = 128
_RB = 1000


def _mesh():
    return plsc.VectorSubcoreMesh(
        core_axis_name="c", subcore_axis_name="s",
        num_cores=_NC, num_subcores=_NS)


def _make_seg_sum():
    out_type = (
        jax.ShapeDtypeStruct((_NP, _HF), jnp.float32),
        jax.ShapeDtypeStruct((_NP, _HF), jnp.float32),
    )
    scratch = (
        pltpu.VMEM_SHARED((_NP, _HF), jnp.float32),
        pltpu.VMEM((_ZR, _HF), jnp.float32),
    ) + 3 * (
        pltpu.VMEM((_CH,), jnp.int32),
        pltpu.VMEM((_CH,), jnp.int32),
        pltpu.VMEM((_CH, _HF), jnp.float32),
        pltpu.SemaphoreType.DMA,
        pltpu.SemaphoreType.DMA,
    )

    def body(src_h, dst_h, tlo_h, thi_h, z_h, slo_h, shi_h,
             acc, zbuf, *bufrefs):
        cid = lax.axis_index("c")
        sid = lax.axis_index("s")
        r0 = sid * _RPS

        pltpu.sync_copy(z_h, zbuf)
        for j in range(_RPS // _ZR):
            pltpu.sync_copy(zbuf, acc.at[pl.ds(r0 + j * _ZR, _ZR)])
        plsc.subcore_barrier()

        e0 = sid * _EPS
        bufs = tuple(tuple(bufrefs[5 * i:5 * i + 5]) for i in range(3))

        def run(tab_h):
            def start(b, c):
                sv, dv, rv, sm, _ = b
                off = e0 + c * _CH
                pltpu.sync_copy(src_h.at[pl.ds(off, _CH)], sv)
                pltpu.sync_copy(dst_h.at[pl.ds(off, _CH)], dv)
                pltpu.async_copy(tab_h.at[sv], rv, sm)

            def finish(b):
                sv, dv, rv, sm, ssm = b
                pltpu.make_async_copy(tab_h.at[sv], rv, sm).wait()
                pltpu.async_copy(rv, acc.at[dv], ssm, add=True)

            def drain(b):
                sv, dv, rv, sm, ssm = b
                pltpu.make_async_copy(rv, acc.at[dv], ssm).wait()

            for i in range(3):
                start(bufs[i], i)

            def body3(c3, carry):
                for i in range(3):
                    b = bufs[i]
                    c = c3 * 3 + i
                    finish(b)
                    nxt = c + 3

                    @pl.when(nxt < _NCHUNK)
                    def _():
                        drain(b)
                        start(b, nxt)
                return carry
            lax.fori_loop(0, _NCHUNK // 3, body3, 0)
            for c in range(3 * (_NCHUNK // 3), _NCHUNK):
                finish(bufs[c % 3])
            for i in range(3):
                drain(bufs[i])

        @pl.when(cid == 0)
        def _():
            run(tlo_h)

        @pl.when(cid == 1)
        def _():
            run(thi_h)

        plsc.subcore_barrier()

        @pl.when(cid == 0)
        def _():
            pltpu.sync_copy(acc.at[pl.ds(r0, _RPS)], slo_h.at[pl.ds(r0, _RPS)])

        @pl.when(cid == 1)
        def _():
            pltpu.sync_copy(acc.at[pl.ds(r0, _RPS)], shi_h.at[pl.ds(r0, _RPS)])

    return pl.kernel(body, out_type=out_type, mesh=_mesh(),
                     scratch_types=scratch)


def _make_count():
    out_type = (
        jax.ShapeDtypeStruct((_NP, _HF), jnp.float32),
        jax.ShapeDtypeStruct((_NP, _HF), jnp.float32),
    )
    scratch = (
        pltpu.VMEM_SHARED((_NP, _HF), jnp.float32),
        pltpu.VMEM((_ZR, _HF), jnp.float32),
        pltpu.VMEM((_CHC, _HF), jnp.float32),
    ) + 3 * (
        pltpu.VMEM((_CHC,), jnp.int32),
        pltpu.SemaphoreType.DMA,
    )

    def body(dst_h, z_h, o_h, p0_h, p1_h, acc, zbuf, ones_v, *bufrefs):
        cid = lax.axis_index("c")
        sid = lax.axis_index("s")
        r0 = sid * _RPS

        pltpu.sync_copy(z_h, zbuf)
        for j in range(_RPS // _ZR):
            pltpu.sync_copy(zbuf, acc.at[pl.ds(r0 + j * _ZR, _ZR)])
        pltpu.sync_copy(o_h, ones_v)
        plsc.subcore_barrier()

        e0 = (cid * _NS + sid) * _EPW
        bufs = tuple(tuple(bufrefs[2 * i:2 * i + 2]) for i in range(3))

        def chunkop(b, c):
            dv, sm = b
            off = e0 + c * _CHC
            pltpu.sync_copy(dst_h.at[pl.ds(off, _CHC)], dv)
            pltpu.async_copy(ones_v, acc.at[dv], sm, add=True)

        def drain(b):
            dv, sm = b
            pltpu.make_async_copy(ones_v, acc.at[dv], sm).wait()

        for i in range(3):
            chunkop(bufs[i], i)

        def body3(c3, carry):
            for i in range(3):
                b = bufs[i]
                c = c3 * 3 + i
                nxt = c + 3

                @pl.when(nxt < _NCHUNKC)
                def _():
                    drain(b)
                    chunkop(b, nxt)
            return carry
        lax.fori_loop(0, _NCHUNKC // 3 + 1, body3, 0)
        for i in range(3):
            drain(bufs[i])

        plsc.subcore_barrier()

        @pl.when(cid == 0)
        def _():
            pltpu.sync_copy(acc.at[pl.ds(r0, _RPS)], p0_h.at[pl.ds(r0, _RPS)])

        @pl.when(cid == 1)
        def _():
            pltpu.sync_copy(acc.at[pl.ds(r0, _RPS)], p1_h.at[pl.ds(r0, _RPS)])

    return pl.kernel(body, out_type=out_type, mesh=_mesh(),
                     scratch_types=scratch)


def _dense_body(relu, slo, shi, p0, p1, xl, xh, wl, wr, b, olo, ohi):
    c = jnp.maximum(p0[:, 0:1] + p1[:, 0:1], 1.0)
    s = jnp.concatenate([slo[...], shi[...]], axis=1)
    xx = jnp.concatenate([xl[...], xh[...]], axis=1)
    y = jnp.dot(s / c, wl[...], preferred_element_type=jnp.float32)
    y = y + jnp.dot(xx, wr[...], preferred_element_type=jnp.float32)
    y = y + b[0:1, :]
    if relu:
        y = jnp.maximum(y, 0.0)
    olo[...] = y[:, :_HF]
    ohi[...] = y[:, _HF:]


def _dense(relu, slo, shi, p0, p1, xl, xh, wl, wr, b):
    rowspec = pl.BlockSpec((_RB, _HF), lambda i: (i, 0))
    return pl.pallas_call(
        functools.partial(_dense_body, relu),
        grid=(_N // _RB,),
        in_specs=[rowspec, rowspec, rowspec, rowspec, rowspec, rowspec,
                  pl.BlockSpec((_F, _F), lambda i: (0, 0)),
                  pl.BlockSpec((_F, _F), lambda i: (0, 0)),
                  pl.BlockSpec((8, _F), lambda i: (0, 0))],
        out_specs=[rowspec, rowspec],
        out_shape=[jax.ShapeDtypeStruct((_N, _HF), jnp.float32)] * 2,
    )(slo, shi, p0, p1, xl, xh, wl, wr, b)


def kernel(x, edge_index, W1_l, b1_l, W1_r,
           Wmu_l, bmu_l, Wmu_r, Wls_l, bls_l, Wls_r):
    src = edge_index[0].astype(jnp.int32)
    dst = edge_index[1].astype(jnp.int32)
    xlo = x[:, :_HF]
    xhi = x[:, _HF:]
    z = jnp.zeros((_ZR, _HF), jnp.float32)
    ones = jnp.ones((_CHC, _HF), jnp.float32)

    seg = _make_seg_sum()
    count = _make_count()

    p0, p1 = count(dst, z, ones)
    slo, shi = seg(src, dst, xlo, xhi, z)
    b1 = jnp.tile(b1_l[None, :], (8, 1))
    hlo, hhi = _dense(True, slo, shi, p0, p1, xlo, xhi, W1_l.T, W1_r.T, b1)

    slo2, shi2 = seg(src, dst, hlo, hhi, z)
    wl2 = jnp.concatenate([Wmu_l.T, Wls_l.T], axis=1)
    wr2 = jnp.concatenate([Wmu_r.T, Wls_r.T], axis=1)
    b2 = jnp.tile(jnp.concatenate([bmu_l, bls_l])[None, :], (8, 1))
    mu, ls = _dense(False, slo2, shi2, p0, p1, hlo, hhi, wl2, wr2, b2)
    return (mu, ls)
